# baseline (speedup 1.0000x reference)
import functools

import jax
import jax.numpy as jnp
from jax import lax
from jax.experimental import pallas as pl
from jax.experimental.pallas import tpu as pltpu
from jax.experimental.pallas import tpu_sc as plsc


def kernel(inputs, word_table, pos_table):
    B, S = inputs.shape
    V, D = word_table.shape
    info = plsc.get_sparse_core_info()
    NC, NS, L = info.num_cores, info.num_subcores, info.num_lanes
    NW = NC * NS
    assert B % NW == 0 and D % L == 0 and (S * D) % 8 == 0
    seqs_per_w = B // NW

    idx_flat = inputs.reshape(B * S)
    mesh = plsc.VectorSubcoreMesh(core_axis_name="c", subcore_axis_name="s")

    @functools.partial(
        pl.kernel,
        out_type=jax.ShapeDtypeStruct((B * S, D), jnp.float32),
        mesh=mesh,
        scratch_types=[
            pltpu.VMEM((S,), jnp.int32),
            pltpu.VMEM((S, D), jnp.float32),
            pltpu.VMEM((S, D), jnp.float32),
            pltpu.SemaphoreType.DMA,
        ],
        compiler_params=pltpu.CompilerParams(use_tc_tiling_on_sc=False),
    )
    def emb_kernel(idx_hbm, table_hbm, pos_hbm, out_hbm, idx_v, rows_v, pos_v, gsem):
        wid = lax.axis_index("s") * NC + lax.axis_index("c")
        base = wid * seqs_per_w * S
        pltpu.sync_copy(pos_hbm, pos_v)

        def body(b, carry):
            start = base + b * S
            pltpu.sync_copy(idx_hbm.at[pl.ds(start, S)], idx_v)
            pltpu.async_copy(table_hbm.at[idx_v], rows_v, gsem).wait()

            def add_row(srow, c2):
                for j in range(D // L):
                    sl = pl.ds(j * L, L)
                    rows_v[srow, sl] = rows_v[srow, sl] + pos_v[srow, sl]
                return c2

            lax.fori_loop(0, S, add_row, 0)
            pltpu.sync_copy(rows_v, out_hbm.at[pl.ds(start, S)])
            return carry

        lax.fori_loop(0, seqs_per_w, body, 0)

    out = emb_kernel(idx_flat, word_table, pos_table)
    return out.reshape(B, S, D)
